# Initial kernel scaffold; baseline (speedup 1.0000x reference)
import jax
import jax.numpy as jnp
from jax.experimental import pallas as pl


def kernel(x, edge_index, params):
    raise NotImplementedError("write your pallas kernel here")


# speedup vs baseline: 1.3503x; 1.3503x over previous
import functools

import numpy as np
import jax
import jax.numpy as jnp
from jax import lax
from jax.experimental import pallas as pl
from jax.experimental.pallas import tpu as pltpu
from jax.experimental.pallas import tpu_sc as plsc

H = 4
AW = 16
CH = 128
NTILES = 32
F32 = jnp.float32
I32 = jnp.int32


def _ru(a, b):
    return -(-a // b) * b


def _tc_prep(xp, lin_w, u_w, c):
    R, Fin = xp.shape
    HO = lin_w.shape[1]
    O = HO // H

    def body(x_ref, w_ref, u_ref, c_ref, y_ref, ab_ref, sa_ref):
        xv = x_ref[...]
        xw = jnp.dot(xv, w_ref[...], preferred_element_type=F32)
        xu = jnp.dot(xv, u_ref[...], preferred_element_type=F32)
        cc = c_ref[...]
        a = jnp.exp(xu + cc)
        b = jnp.exp(-xu)
        ec = jnp.exp(cc - jnp.max(cc))
        sm = ec / jnp.sum(ec)
        sa = jnp.zeros((R, O), F32)
        for h in range(H):
            blk = xw[:, h * O:(h + 1) * O]
            y_ref[:, h * O:(h + 1) * O] = blk * a[:, h:h + 1]
            sa = sa + blk * sm[:, h:h + 1]
        sa_ref[...] = sa
        ab_ref[:, 0:H] = a
        ab_ref[:, H:2 * H] = b
        ab_ref[:, 2 * H:AW] = jnp.zeros((R, AW - 2 * H), F32)

    return pl.pallas_call(
        body,
        out_shape=[
            jax.ShapeDtypeStruct((R, HO), F32),
            jax.ShapeDtypeStruct((R, AW), F32),
            jax.ShapeDtypeStruct((R, O), F32),
        ],
    )(xp, lin_w, u_w, c.reshape(1, H))


def _tc_post(agg0, agg1, cnt0, cnt1, sa, bias, n_real, relu, bn):
    R, O = sa.shape

    def body(*refs):
        if bn is None:
            a0, a1, c0, c1, s_ref, b_ref, o_ref = refs
        else:
            a0, a1, c0, c1, s_ref, b_ref, g_ref, bb_ref, o_ref = refs
        aggw = a0[:R, :] + a1[:R, :]
        if O == 16:
            agg = aggw
        else:
            agg = (aggw[:, 0:4] + aggw[:, 4:8]
                   + aggw[:, 8:12] + aggw[:, 12:16])
        cnt = c0[:R, 0:1] + c1[:R, 0:1] + 1.0
        val = (agg + s_ref[...]) / cnt + b_ref[...]
        if relu:
            val = jnp.maximum(val, 0.0)
        row = lax.broadcasted_iota(I32, (R, O), 0)
        val = jnp.where(row < n_real, val, 0.0)
        if bn is not None:
            inv_n = 1.0 / n_real
            mu = jnp.sum(val, axis=0, keepdims=True) * inv_n
            ex2 = jnp.sum(val * val, axis=0, keepdims=True) * inv_n
            var = ex2 - mu * mu
            val = g_ref[...] * (val - mu) * lax.rsqrt(var + 1e-5) + bb_ref[...]
            val = jnp.where(row < n_real, val, 0.0)
        o_ref[...] = val

    args = [agg0, agg1, cnt0, cnt1, sa, bias.reshape(1, O)]
    if bn is not None:
        args += [bn[0].reshape(1, O), bn[1].reshape(1, O)]
    return pl.pallas_call(
        body,
        out_shape=jax.ShapeDtypeStruct((R, O), F32),
    )(*args)


def _tc_score(xp, w):
    R = xp.shape[0]

    def body(x_ref, w_ref, o_ref):
        wv = w_ref[...]
        nrm = lax.rsqrt(jnp.sum(wv * wv))
        s = jnp.dot(x_ref[...], wv, preferred_element_type=F32) * nrm
        o_ref[...] = jnp.tanh(s)

    return pl.pallas_call(
        body,
        out_shape=jax.ShapeDtypeStruct((R, 1), F32),
    )(xp, w)


def _tc_mlp(xp, p1, p2, p3, po, n_real):
    def body(x_ref, w1, b1, w2, b2, w3, b3, w4, b4, o_ref):
        z = x_ref[...]
        z = jnp.maximum(jnp.dot(z, w1[...], preferred_element_type=F32) + b1[...], 0.0)
        z = jnp.maximum(jnp.dot(z, w2[...], preferred_element_type=F32) + b2[...], 0.0)
        z = jnp.maximum(jnp.dot(z, w3[...], preferred_element_type=F32) + b3[...], 0.0)
        t = jnp.dot(z, w4[...], preferred_element_type=F32) + b4[...]
        o_ref[...] = (1.0 / (1.0 + jnp.exp(-t)))[:n_real, :]

    return pl.pallas_call(
        body,
        out_shape=jax.ShapeDtypeStruct((n_real, 1), F32),
    )(xp,
      p1['w'], p1['b'].reshape(1, -1),
      p2['w'], p2['b'].reshape(1, -1),
      p3['w'], p3['b'].reshape(1, -1),
      po['w'], po['b'].reshape(1, -1))


_MESH = plsc.VectorSubcoreMesh(core_axis_name="c", subcore_axis_name="s")
_SC_PARAMS = pltpu.CompilerParams(use_tc_tiling_on_sc=False,
                                  needs_layout_passes=False)


def _sc_edge(y, ab, srcp, dstp, np_rows, O):
    R, HO = y.shape
    Ep = srcp.shape[0]
    cpt = Ep // CH // NTILES
    rps = np_rows // 16

    @functools.partial(
        pl.kernel, mesh=_MESH, compiler_params=_SC_PARAMS,
        out_type=[
            jax.ShapeDtypeStruct((np_rows, AW), F32),
            jax.ShapeDtypeStruct((np_rows, AW), F32),
        ],
        scratch_types=[
            pltpu.VMEM_SHARED((np_rows, AW), F32),
            pltpu.VMEM((CH,), I32),
            pltpu.VMEM((CH,), I32),
            pltpu.VMEM((CH, HO), F32),
            pltpu.VMEM((CH, AW), F32),
            pltpu.VMEM((CH, AW), F32),
            pltpu.VMEM((CH, AW), F32),
            pltpu.VMEM((rps, AW), F32),
            pltpu.SemaphoreType.DMA,
        ],
    )
    def k(y_h, ab_h, s_h, d_h, o0, o1, acc, sv, dv, yv, av, bv, mv, zv, sem):
        cid = lax.axis_index("c")
        sid = lax.axis_index("s")
        wid = cid * 16 + sid
        z16 = jnp.zeros((16,), F32)
        iot = lax.iota(I32, 16)

        def zz(i, _):
            zv[i, :] = z16
            return 0
        lax.fori_loop(0, rps, zz, 0)
        pltpu.sync_copy(zv, acc.at[pl.ds(sid * rps, rps)])
        plsc.subcore_barrier()

        if O == 16:
            def edge_body(e, _):
                arow = av[e, :]
                brow = bv[e, :]
                num = z16
                den = z16
                for h in range(H):
                    bd = jnp.full((16,), brow[H + h], F32)
                    as_ = jnp.full((16,), arow[h], F32)
                    ys = yv[e, pl.ds(h * 16, 16)]
                    num = num + bd * ys
                    den = den + bd * as_
                mv[e, :] = num / den
                return 0
        else:
            def edge_body(e, _):
                arow = av[e, :]
                brow = bv[e, :]
                den = z16
                bds = []
                for h in range(H):
                    bd = jnp.full((16,), brow[H + h], F32)
                    as_ = jnp.full((16,), arow[h], F32)
                    den = den + bd * as_
                    bds.append(bd)
                bd16 = jnp.where(iot < 4, bds[0],
                                 jnp.where(iot < 8, bds[1],
                                           jnp.where(iot < 12, bds[2], bds[3])))
                mv[e, :] = yv[e, :] * bd16 / den
                return 0
        n_inner = CH

        def chunk_body(ci, _):
            base = (wid * cpt + ci) * CH
            pltpu.sync_copy(s_h.at[pl.ds(base, CH)], sv)
            pltpu.sync_copy(d_h.at[pl.ds(base, CH)], dv)
            pltpu.async_copy(y_h.at[sv], yv, sem).wait()
            pltpu.async_copy(ab_h.at[sv], av, sem).wait()
            pltpu.async_copy(ab_h.at[dv], bv, sem).wait()
            lax.fori_loop(0, n_inner, edge_body, 0)
            pltpu.sync_copy(mv, acc.at[dv], add=True)
            return 0
        lax.fori_loop(0, cpt, chunk_body, 0)
        plsc.subcore_barrier()

        @pl.when(cid == 0)
        def _():
            pltpu.sync_copy(acc.at[pl.ds(sid * rps, rps)],
                            o0.at[pl.ds(sid * rps, rps)])

        @pl.when(cid == 1)
        def _():
            pltpu.sync_copy(acc.at[pl.ds(sid * rps, rps)],
                            o1.at[pl.ds(sid * rps, rps)])

    return k(y, ab, srcp, dstp)


def _sc_cnt(dstp, np_rows):
    Ep = dstp.shape[0]
    cpt = Ep // CH // NTILES
    rps = np_rows // 16

    @functools.partial(
        pl.kernel, mesh=_MESH, compiler_params=_SC_PARAMS,
        out_type=[
            jax.ShapeDtypeStruct((np_rows, AW), F32),
            jax.ShapeDtypeStruct((np_rows, AW), F32),
        ],
        scratch_types=[
            pltpu.VMEM_SHARED((np_rows, AW), F32),
            pltpu.VMEM((CH,), I32),
            pltpu.VMEM((CH, AW), F32),
            pltpu.VMEM((rps, AW), F32),
        ],
    )
    def k(d_h, o0, o1, acc, dv, ov, zv):
        cid = lax.axis_index("c")
        sid = lax.axis_index("s")
        wid = cid * 16 + sid
        z16 = jnp.zeros((16,), F32)
        o16 = jnp.ones((16,), F32)

        def fill_o(i, _):
            ov[i, :] = o16
            return 0
        lax.fori_loop(0, CH, fill_o, 0)

        def fill_z(i, _):
            zv[i, :] = z16
            return 0
        lax.fori_loop(0, rps, fill_z, 0)
        pltpu.sync_copy(zv, acc.at[pl.ds(sid * rps, rps)])
        plsc.subcore_barrier()

        def chunk_body(ci, _):
            base = (wid * cpt + ci) * CH
            pltpu.sync_copy(d_h.at[pl.ds(base, CH)], dv)
            pltpu.sync_copy(ov, acc.at[dv], add=True)
            return 0
        lax.fori_loop(0, cpt, chunk_body, 0)
        plsc.subcore_barrier()

        @pl.when(cid == 0)
        def _():
            pltpu.sync_copy(acc.at[pl.ds(sid * rps, rps)],
                            o0.at[pl.ds(sid * rps, rps)])

        @pl.when(cid == 1)
        def _():
            pltpu.sync_copy(acc.at[pl.ds(sid * rps, rps)],
                            o1.at[pl.ds(sid * rps, rps)])

    return k(dstp)


def _sc_pool(xp, permp, valsp, srcp, dstp, n_old, k_new):
    KP = permp.shape[0]
    Ep = srcp.shape[0]
    kc_total = KP // CH
    kcpt = kc_total // NTILES
    ecpt = Ep // CH // NTILES
    NT = _ru(n_old + 1, 16)

    @functools.partial(
        pl.kernel, mesh=_MESH, compiler_params=_SC_PARAMS,
        out_type=[
            jax.ShapeDtypeStruct((KP, 16), F32),
            jax.ShapeDtypeStruct((Ep,), I32),
            jax.ShapeDtypeStruct((Ep,), I32),
        ],
        scratch_types=[
            pltpu.VMEM((NT,), I32),
            pltpu.VMEM((CH,), I32),
            pltpu.VMEM((CH,), F32),
            pltpu.VMEM((CH, 16), F32),
            pltpu.VMEM((CH,), I32),
            pltpu.VMEM((CH,), I32),
            pltpu.VMEM((CH,), I32),
            pltpu.VMEM((CH,), I32),
            pltpu.SemaphoreType.DMA,
        ],
    )
    def k(x_h, p_h, v_h, s_h, d_h, xo, so, do,
          inv, pv, vv, xg, sv, dv, nsv, ndv, sem):
        cid = lax.axis_index("c")
        sid = lax.axis_index("s")
        wid = cid * 16 + sid
        iot = lax.iota(I32, 16)
        kfull = jnp.full((16,), k_new, I32)

        def init_body(i, _):
            inv[pl.ds(i * 16, 16)] = kfull
            return 0
        lax.fori_loop(0, NT // 16, init_body, 0)

        def perm_chunk(ci, _):
            pltpu.sync_copy(p_h.at[pl.ds(ci * CH, CH)], pv)

            def g_body(g, _):
                pvals = pv[pl.ds(g * 16, 16)]
                rv = jnp.full((16,), ci * CH + g * 16, I32) + iot
                plsc.store_scatter(inv, [pvals], rv)
                return 0
            lax.fori_loop(0, CH // 16, g_body, 0)
            return 0
        lax.fori_loop(0, kc_total, perm_chunk, 0)
        plsc.store_scatter(inv, [jnp.full((16,), n_old, I32)], kfull)

        def row_chunk(ci, _):
            base = (wid * kcpt + ci) * CH
            pltpu.sync_copy(p_h.at[pl.ds(base, CH)], pv)
            pltpu.sync_copy(v_h.at[pl.ds(base, CH)], vv)
            pltpu.async_copy(x_h.at[pv], xg, sem).wait()

            def r_body(r, _):
                s = plsc.load_gather(vv, [jnp.full((16,), r, I32)])
                xg[r, :] = xg[r, :] * s
                return 0
            lax.fori_loop(0, CH, r_body, 0)
            pltpu.sync_copy(xg, xo.at[pl.ds(base, CH)])
            return 0
        lax.fori_loop(0, kcpt, row_chunk, 0)

        def e_chunk(ci, _):
            base = (wid * ecpt + ci) * CH
            pltpu.sync_copy(s_h.at[pl.ds(base, CH)], sv)
            pltpu.sync_copy(d_h.at[pl.ds(base, CH)], dv)

            def g_body(g, _):
                svals = sv[pl.ds(g * 16, 16)]
                dvals = dv[pl.ds(g * 16, 16)]
                ns = plsc.load_gather(inv, [svals])
                nd = plsc.load_gather(inv, [dvals])
                keep = jnp.logical_and(ns != kfull, nd != kfull)
                nsv[pl.ds(g * 16, 16)] = jnp.where(keep, ns, kfull)
                ndv[pl.ds(g * 16, 16)] = jnp.where(keep, nd, kfull)
                return 0
            lax.fori_loop(0, CH // 16, g_body, 0)
            pltpu.sync_copy(nsv, so.at[pl.ds(base, CH)])
            pltpu.sync_copy(ndv, do.at[pl.ds(base, CH)])
            return 0
        lax.fori_loop(0, ecpt, e_chunk, 0)

    return k(xp, permp, valsp, srcp, dstp)


def _conv(xp, srcp, dstp, cnt0, cnt1, p, n_real, np_rows, relu, bn=None):
    y, ab, sa = _tc_prep(xp, p['lin_w'], p['u_w'], p['c'])
    O = p['lin_w'].shape[1] // H
    agg0, agg1 = _sc_edge(y, ab, srcp, dstp, np_rows, O)
    return _tc_post(agg0, agg1, cnt0, cnt1, sa, p['bias'], n_real, relu, bn)


def kernel(x, edge_index, params):
    N, F = x.shape
    E = edge_index.shape[1]
    src = edge_index[0].astype(I32)
    dst = edge_index[1].astype(I32)

    Ep = _ru(E, NTILES * CH)
    pad_e = Ep - E
    srcp = jnp.concatenate([src, jnp.full((pad_e,), N, I32)])
    dstp = jnp.concatenate([dst, jnp.full((pad_e,), N, I32)])
    xp = jnp.concatenate([x, jnp.zeros((1, F), F32)], axis=0)

    n = N
    for bname in ('b1', 'b2', 'b3'):
        p = params[bname]
        np_rows = _ru(n + 1, 128)
        cnt0, cnt1 = _sc_cnt(dstp, np_rows)
        xp = _conv(xp, srcp, dstp, cnt0, cnt1, p['c1'], n, np_rows, True)
        xp = _conv(xp, srcp, dstp, cnt0, cnt1, p['c2'], n, np_rows, True)
        xp = _conv(xp, srcp, dstp, cnt0, cnt1, p['c3'], n, np_rows, False)

        sc = _tc_score(xp, p['pool_w'].reshape(16, 1))
        scores = sc[:n, 0]
        k_new = int(np.ceil(0.5 * n))
        vals, perm = lax.top_k(scores, k_new)
        KP = _ru(k_new + 1, NTILES * CH)
        permp = jnp.concatenate([perm.astype(I32),
                                 jnp.full((KP - k_new,), n, I32)])
        valsp = jnp.concatenate([vals, jnp.zeros((KP - k_new,), F32)])
        xnew, srcp, dstp = _sc_pool(xp, permp, valsp, srcp, dstp, n, k_new)
        n = k_new
        xp = xnew[:n + 1]
        np_rows = _ru(n + 1, 128)
        cnt0, cnt1 = _sc_cnt(dstp, np_rows)
        xp = _conv(xp, srcp, dstp, cnt0, cnt1, p['c4'], n, np_rows, True,
                   bn=(p['bn_g'], p['bn_b']))

    for bname in ('b4', 'b5'):
        p = params[bname]
        xp = _conv(xp, srcp, dstp, cnt0, cnt1, p['c1'], n, np_rows, True)
        xp = _conv(xp, srcp, dstp, cnt0, cnt1, p['c2'], n, np_rows, True)
        xp = _conv(xp, srcp, dstp, cnt0, cnt1, p['c3'], n, np_rows, True)
        xp = _conv(xp, srcp, dstp, cnt0, cnt1, p['c4'], n, np_rows, True,
                   bn=(p['bn_g'], p['bn_b']))

    return _tc_mlp(xp, params['lin1'], params['lin2'], params['lin3'],
                   params['out'], n)
